# Initial kernel scaffold; baseline (speedup 1.0000x reference)
import functools

import jax
import jax.numpy as jnp
from jax import lax
from jax.experimental import pallas as pl
from jax.experimental.pallas import tpu as pltpu
from jax.experimental.pallas import tpu_sc as plsc

N_NODES = 10000
N_EDGES = 320000
D = 128
NPAD = 10240
NC = 2
NS = 16
NW = NC * NS
EPS = N_EDGES // NW
K = 80
ITERS = EPS // K
RPS = NPAD // NS
DEGW = 16

BLK = 1024
GRID = NPAD // BLK

_MESH = plsc.VectorSubcoreMesh(
    core_axis_name="c", subcore_axis_name="s", num_cores=NC, num_subcores=NS)


def _deg_body(src_hbm, dst_hbm, out_hbm, idx2, ones_v, zb, dego, degi, sem):
    c = lax.axis_index("c")
    s = lax.axis_index("s")
    zv = jnp.zeros((16,), jnp.float32)
    ov = jnp.ones((16,), jnp.float32)
    for i in range(16):
        zb[i, :] = zv
        ones_v[i, :] = ov
    for i in range(1, K // 16):
        pltpu.sync_copy(ones_v.at[pl.ds(0, 16)], ones_v.at[pl.ds(i * 16, 16)])
    def zloop(i, _):
        pltpu.sync_copy(zb, dego.at[pl.ds(s * RPS + i * 16, 16)])
        pltpu.sync_copy(zb, degi.at[pl.ds(s * RPS + i * 16, 16)])
        return 0
    lax.fori_loop(0, RPS // 16, zloop, 0)
    plsc.subcore_barrier()

    base = (c * NS + s) * EPS
    def body(i, _):
        off = base + i * K
        pltpu.sync_copy(src_hbm.at[pl.ds(off, K)], idx2.at[0])
        pltpu.sync_copy(dst_hbm.at[pl.ds(off, K)], idx2.at[1])
        pltpu.sync_copy(ones_v, dego.at[idx2.at[0]], add=True)
        pltpu.sync_copy(ones_v, degi.at[idx2.at[1]], add=True)
        return 0
    lax.fori_loop(0, ITERS, body, 0)
    plsc.subcore_barrier()

    r0 = s * RPS
    pltpu.sync_copy(dego.at[pl.ds(r0, RPS)], out_hbm.at[c, 0, pl.ds(r0, RPS)])
    pltpu.sync_copy(degi.at[pl.ds(r0, RPS)], out_hbm.at[c, 1, pl.ds(r0, RPS)])


_deg_call = pl.kernel(
    _deg_body,
    out_type=jax.ShapeDtypeStruct((NC, 2, NPAD, DEGW), jnp.float32),
    mesh=_MESH,
    scratch_types=[
        pltpu.VMEM((2, K), jnp.int32),
        pltpu.VMEM((K, DEGW), jnp.float32),
        pltpu.VMEM((16, DEGW), jnp.float32),
        pltpu.VMEM_SHARED((NPAD, DEGW), jnp.float32),
        pltpu.VMEM_SHARED((NPAD, DEGW), jnp.float32),
        pltpu.SemaphoreType.DMA,
    ],
)


def _spmm_body(src_hbm, dst_hbm, tbl_hbm, out_hbm, idx2, rows, zb, agg, sem):
    c = lax.axis_index("c")
    s = lax.axis_index("s")
    zv = jnp.zeros((16,), jnp.float32)
    for i in range(16):
        for j in range(D // 16):
            zb[i, pl.ds(j * 16, 16)] = zv
    def zloop(i, _):
        pltpu.sync_copy(zb, agg.at[pl.ds(s * RPS + i * 16, 16)])
        return 0
    lax.fori_loop(0, RPS // 16, zloop, 0)
    plsc.subcore_barrier()

    base = (c * NS + s) * EPS
    def body(i, _):
        off = base + i * K
        pltpu.sync_copy(src_hbm.at[pl.ds(off, K)], idx2.at[0])
        pltpu.sync_copy(dst_hbm.at[pl.ds(off, K)], idx2.at[1])
        pltpu.async_copy(tbl_hbm.at[idx2.at[0]], rows, sem).wait()
        pltpu.sync_copy(rows, agg.at[idx2.at[1]], add=True)
        return 0
    lax.fori_loop(0, ITERS, body, 0)
    plsc.subcore_barrier()

    r0 = s * RPS
    pltpu.sync_copy(agg.at[pl.ds(r0, RPS)], out_hbm.at[c, pl.ds(r0, RPS)])


_spmm_call = pl.kernel(
    _spmm_body,
    out_type=jax.ShapeDtypeStruct((NC, NPAD, D), jnp.float32),
    mesh=_MESH,
    scratch_types=[
        pltpu.VMEM((2, K), jnp.int32),
        pltpu.VMEM((K, D), jnp.float32),
        pltpu.VMEM((16, D), jnp.float32),
        pltpu.VMEM_SHARED((NPAD, D), jnp.float32),
        pltpu.SemaphoreType.DMA,
    ],
)


def _prep_body(x_ref, degp_ref, xn_ref, no_ref, ni_ref):
    dp = degp_ref[...]
    dg_o = (dp[0, 0] + dp[1, 0])[:, 0:1]
    dg_i = (dp[0, 1] + dp[1, 1])[:, 0:1]
    no = jnp.where(dg_o > 0, lax.rsqrt(jnp.maximum(dg_o, 1.0)), 0.0)
    ni = jnp.where(dg_i > 0, lax.rsqrt(jnp.maximum(dg_i, 1.0)), 0.0)
    xn_ref[...] = x_ref[...] * no
    no_ref[...] = jnp.broadcast_to(no, (BLK, D))
    ni_ref[...] = jnp.broadcast_to(ni, (BLK, D))


def _prep(x_pad, degp):
    return pl.pallas_call(
        _prep_body,
        grid=(GRID,),
        in_specs=[
            pl.BlockSpec((BLK, D), lambda i: (i, 0)),
            pl.BlockSpec((NC, 2, BLK, DEGW), lambda i: (0, 0, i, 0)),
        ],
        out_specs=[
            pl.BlockSpec((BLK, D), lambda i: (i, 0)),
            pl.BlockSpec((BLK, D), lambda i: (i, 0)),
            pl.BlockSpec((BLK, D), lambda i: (i, 0)),
        ],
        out_shape=[
            jax.ShapeDtypeStruct((NPAD, D), jnp.float32),
            jax.ShapeDtypeStruct((NPAD, D), jnp.float32),
            jax.ShapeDtypeStruct((NPAD, D), jnp.float32),
        ],
    )(x_pad, degp)


def _layer_body(aggp_ref, ni_ref, no_ref, w_ref, b_ref, out_ref):
    a = (aggp_ref[0] + aggp_ref[1]) * ni_ref[...]
    h = jnp.dot(a, w_ref[...], preferred_element_type=jnp.float32) + b_ref[...]
    out_ref[...] = jnp.maximum(h, 0.0) * no_ref[...]


def _layer(aggp, ni, no, w, b2d):
    return pl.pallas_call(
        _layer_body,
        grid=(GRID,),
        in_specs=[
            pl.BlockSpec((NC, BLK, D), lambda i: (0, i, 0)),
            pl.BlockSpec((BLK, D), lambda i: (i, 0)),
            pl.BlockSpec((BLK, D), lambda i: (i, 0)),
            pl.BlockSpec((D, D), lambda i: (0, 0)),
            pl.BlockSpec((1, D), lambda i: (0, 0)),
        ],
        out_specs=pl.BlockSpec((BLK, D), lambda i: (i, 0)),
        out_shape=jax.ShapeDtypeStruct((NPAD, D), jnp.float32),
    )(aggp, ni, no, w, b2d)


def _final_body(aggp_ref, ni_ref, w2_ref, b2_ref, w3_ref, b3_ref, out_ref,
                acc_ref):
    i = pl.program_id(0)
    a = (aggp_ref[0] + aggp_ref[1]) * ni_ref[...]
    h = jnp.dot(a, w2_ref[...], preferred_element_type=jnp.float32) + b2_ref[...]
    h = jnp.maximum(h, 0.0)
    rows = lax.broadcasted_iota(jnp.int32, (BLK, 1), 0) + i * BLK
    h = jnp.where(rows < N_NODES, h, 0.0)

    @pl.when(i == 0)
    def _():
        acc_ref[...] = jnp.zeros_like(acc_ref)

    acc_ref[...] += jnp.sum(h, axis=0, keepdims=True)

    @pl.when(i == GRID - 1)
    def _():
        hg = acc_ref[...] * (1.0 / N_NODES)
        out_ref[...] = (
            jnp.dot(hg, w3_ref[...], preferred_element_type=jnp.float32)
            + b3_ref[...])


def _final(aggp, ni, w2, b2d, w3pad, b3pad):
    return pl.pallas_call(
        _final_body,
        grid=(GRID,),
        in_specs=[
            pl.BlockSpec((NC, BLK, D), lambda i: (0, i, 0)),
            pl.BlockSpec((BLK, D), lambda i: (i, 0)),
            pl.BlockSpec((D, D), lambda i: (0, 0)),
            pl.BlockSpec((1, D), lambda i: (0, 0)),
            pl.BlockSpec((D, 8), lambda i: (0, 0)),
            pl.BlockSpec((1, 8), lambda i: (0, 0)),
        ],
        out_specs=pl.BlockSpec((1, 8), lambda i: (0, 0)),
        out_shape=jax.ShapeDtypeStruct((1, 8), jnp.float32),
        scratch_shapes=[pltpu.VMEM((1, D), jnp.float32)],
    )(aggp, ni, w2, b2d, w3pad, b3pad)


def kernel(x, edge_index, W1, b1, W2, b2, W3, b3):
    src = edge_index[0]
    dst = edge_index[1]
    x_pad = jnp.pad(x, ((0, NPAD - x.shape[0]), (0, 0)))

    degp = _deg_call(src, dst)
    xn, no, ni = _prep(x_pad, degp)
    agg1p = _spmm_call(src, dst, xn)
    h1n = _layer(agg1p, ni, no, W1, b1.reshape(1, D))
    agg2p = _spmm_call(src, dst, h1n)
    w3pad = jnp.pad(W3, ((0, 0), (0, 8 - W3.shape[1])))
    b3pad = jnp.pad(b3.reshape(1, -1), ((0, 0), (0, 8 - b3.shape[0])))
    outp = _final(agg2p, ni, W2, b2.reshape(1, D), w3pad, b3pad)
    return outp[:, :1]


# speedup vs baseline: 4.3818x; 4.3818x over previous
import functools

import jax
import jax.numpy as jnp
from jax import lax
from jax.experimental import pallas as pl
from jax.experimental.pallas import tpu as pltpu
from jax.experimental.pallas import tpu_sc as plsc

N_NODES = 10000
N_EDGES = 320000
D = 128
NPAD = 10240
NC = 2
NS = 16
NW = NC * NS
EPS = N_EDGES // NW
K = 80
ITERS = EPS // K
RPS = NPAD // NS

BLK = 1024
GRID = NPAD // BLK

_MESH = plsc.VectorSubcoreMesh(
    core_axis_name="c", subcore_axis_name="s", num_cores=NC, num_subcores=NS)


def _deg_body(src_hbm, dst_hbm, out_hbm, idx2, ones_o, ones_i, zb, deg, sem):
    c = lax.axis_index("c")
    s = lax.axis_index("s")
    zv = jnp.zeros((16,), jnp.float32)
    ov = jnp.ones((16,), jnp.float32)
    for i in range(16):
        for j in range(D // 16):
            zb[i, pl.ds(j * 16, 16)] = zv
    for i in range(K):
        for j in range(D // 16):
            left = j < (D // 32)
            ones_o[i, pl.ds(j * 16, 16)] = ov if left else zv
            ones_i[i, pl.ds(j * 16, 16)] = zv if left else ov
    def zloop(i, _):
        pltpu.sync_copy(zb, deg.at[pl.ds(s * RPS + i * 16, 16)])
        return 0
    lax.fori_loop(0, RPS // 16, zloop, 0)
    plsc.subcore_barrier()

    base = (c * NS + s) * EPS
    def body(i, _):
        off = base + i * K
        pltpu.sync_copy(src_hbm.at[pl.ds(off, K)], idx2.at[0])
        pltpu.sync_copy(dst_hbm.at[pl.ds(off, K)], idx2.at[1])
        pltpu.sync_copy(ones_o, deg.at[idx2.at[0]], add=True)
        pltpu.sync_copy(ones_i, deg.at[idx2.at[1]], add=True)
        return 0
    lax.fori_loop(0, ITERS, body, 0)
    plsc.subcore_barrier()

    r0 = s * RPS
    pltpu.sync_copy(deg.at[pl.ds(r0, RPS)], out_hbm.at[c, pl.ds(r0, RPS)])


_deg_call = pl.kernel(
    _deg_body,
    out_type=jax.ShapeDtypeStruct((NC, NPAD, D), jnp.float32),
    mesh=_MESH,
    scratch_types=[
        pltpu.VMEM((2, K), jnp.int32),
        pltpu.VMEM((K, D), jnp.float32),
        pltpu.VMEM((K, D), jnp.float32),
        pltpu.VMEM((16, D), jnp.float32),
        pltpu.VMEM_SHARED((NPAD, D), jnp.float32),
        pltpu.SemaphoreType.DMA,
    ],
)


def _spmm_body(src_hbm, dst_hbm, tbl_hbm, out_hbm, idx2, rows, zb, agg, sem):
    c = lax.axis_index("c")
    s = lax.axis_index("s")
    zv = jnp.zeros((16,), jnp.float32)
    for i in range(16):
        for j in range(D // 16):
            zb[i, pl.ds(j * 16, 16)] = zv
    def zloop(i, _):
        pltpu.sync_copy(zb, agg.at[pl.ds(s * RPS + i * 16, 16)])
        return 0
    lax.fori_loop(0, RPS // 16, zloop, 0)
    plsc.subcore_barrier()

    base = (c * NS + s) * EPS
    def body(i, _):
        off = base + i * K
        pltpu.sync_copy(src_hbm.at[pl.ds(off, K)], idx2.at[0])
        pltpu.sync_copy(dst_hbm.at[pl.ds(off, K)], idx2.at[1])
        pltpu.async_copy(tbl_hbm.at[idx2.at[0]], rows, sem).wait()
        pltpu.sync_copy(rows, agg.at[idx2.at[1]], add=True)
        return 0
    lax.fori_loop(0, ITERS, body, 0)
    plsc.subcore_barrier()

    r0 = s * RPS
    pltpu.sync_copy(agg.at[pl.ds(r0, RPS)], out_hbm.at[c, pl.ds(r0, RPS)])


_spmm_call = pl.kernel(
    _spmm_body,
    out_type=jax.ShapeDtypeStruct((NC, NPAD, D), jnp.float32),
    mesh=_MESH,
    scratch_types=[
        pltpu.VMEM((2, K), jnp.int32),
        pltpu.VMEM((K, D), jnp.float32),
        pltpu.VMEM((16, D), jnp.float32),
        pltpu.VMEM_SHARED((NPAD, D), jnp.float32),
        pltpu.SemaphoreType.DMA,
    ],
)


def _prep_body(x_ref, degp_ref, xn_ref, no_ref, ni_ref):
    dp = degp_ref[0] + degp_ref[1]
    dg_o = dp[:, 0:1]
    dg_i = dp[:, 64:65]
    no = jnp.where(dg_o > 0, lax.rsqrt(jnp.maximum(dg_o, 1.0)), 0.0)
    ni = jnp.where(dg_i > 0, lax.rsqrt(jnp.maximum(dg_i, 1.0)), 0.0)
    xn_ref[...] = x_ref[...] * no
    no_ref[...] = jnp.broadcast_to(no, (BLK, D))
    ni_ref[...] = jnp.broadcast_to(ni, (BLK, D))


def _prep(x_pad, degp):
    return pl.pallas_call(
        _prep_body,
        grid=(GRID,),
        in_specs=[
            pl.BlockSpec((BLK, D), lambda i: (i, 0)),
            pl.BlockSpec((NC, BLK, D), lambda i: (0, i, 0)),
        ],
        out_specs=[
            pl.BlockSpec((BLK, D), lambda i: (i, 0)),
            pl.BlockSpec((BLK, D), lambda i: (i, 0)),
            pl.BlockSpec((BLK, D), lambda i: (i, 0)),
        ],
        out_shape=[
            jax.ShapeDtypeStruct((NPAD, D), jnp.float32),
            jax.ShapeDtypeStruct((NPAD, D), jnp.float32),
            jax.ShapeDtypeStruct((NPAD, D), jnp.float32),
        ],
    )(x_pad, degp)


def _layer_body(aggp_ref, ni_ref, no_ref, w_ref, b_ref, out_ref):
    a = (aggp_ref[0] + aggp_ref[1]) * ni_ref[...]
    h = jnp.dot(a, w_ref[...], preferred_element_type=jnp.float32) + b_ref[...]
    out_ref[...] = jnp.maximum(h, 0.0) * no_ref[...]


def _layer(aggp, ni, no, w, b2d):
    return pl.pallas_call(
        _layer_body,
        grid=(GRID,),
        in_specs=[
            pl.BlockSpec((NC, BLK, D), lambda i: (0, i, 0)),
            pl.BlockSpec((BLK, D), lambda i: (i, 0)),
            pl.BlockSpec((BLK, D), lambda i: (i, 0)),
            pl.BlockSpec((D, D), lambda i: (0, 0)),
            pl.BlockSpec((1, D), lambda i: (0, 0)),
        ],
        out_specs=pl.BlockSpec((BLK, D), lambda i: (i, 0)),
        out_shape=jax.ShapeDtypeStruct((NPAD, D), jnp.float32),
    )(aggp, ni, no, w, b2d)


def _final_body(aggp_ref, ni_ref, w2_ref, b2_ref, w3_ref, b3_ref, out_ref,
                acc_ref):
    i = pl.program_id(0)
    a = (aggp_ref[0] + aggp_ref[1]) * ni_ref[...]
    h = jnp.dot(a, w2_ref[...], preferred_element_type=jnp.float32) + b2_ref[...]
    h = jnp.maximum(h, 0.0)
    rows = lax.broadcasted_iota(jnp.int32, (BLK, 1), 0) + i * BLK
    h = jnp.where(rows < N_NODES, h, 0.0)

    @pl.when(i == 0)
    def _():
        acc_ref[...] = jnp.zeros_like(acc_ref)

    acc_ref[...] += jnp.sum(h, axis=0, keepdims=True)

    @pl.when(i == GRID - 1)
    def _():
        hg = acc_ref[...] * (1.0 / N_NODES)
        out_ref[...] = (
            jnp.dot(hg, w3_ref[...], preferred_element_type=jnp.float32)
            + b3_ref[...])


def _final(aggp, ni, w2, b2d, w3pad, b3pad):
    return pl.pallas_call(
        _final_body,
        grid=(GRID,),
        in_specs=[
            pl.BlockSpec((NC, BLK, D), lambda i: (0, i, 0)),
            pl.BlockSpec((BLK, D), lambda i: (i, 0)),
            pl.BlockSpec((D, D), lambda i: (0, 0)),
            pl.BlockSpec((1, D), lambda i: (0, 0)),
            pl.BlockSpec((D, 8), lambda i: (0, 0)),
            pl.BlockSpec((1, 8), lambda i: (0, 0)),
        ],
        out_specs=pl.BlockSpec((1, 8), lambda i: (0, 0)),
        out_shape=jax.ShapeDtypeStruct((1, 8), jnp.float32),
        scratch_shapes=[pltpu.VMEM((1, D), jnp.float32)],
    )(aggp, ni, w2, b2d, w3pad, b3pad)


def kernel(x, edge_index, W1, b1, W2, b2, W3, b3):
    src = edge_index[0]
    dst = edge_index[1]
    x_pad = jnp.pad(x, ((0, NPAD - x.shape[0]), (0, 0)))

    degp = _deg_call(src, dst)
    xn, no, ni = _prep(x_pad, degp)
    agg1p = _spmm_call(src, dst, xn)
    h1n = _layer(agg1p, ni, no, W1, b1.reshape(1, D))
    agg2p = _spmm_call(src, dst, h1n)
    w3pad = jnp.pad(W3, ((0, 0), (0, 8 - W3.shape[1])))
    b3pad = jnp.pad(b3.reshape(1, -1), ((0, 0), (0, 8 - b3.shape[0])))
    outp = _final(agg2p, ni, W2, b2.reshape(1, D), w3pad, b3pad)
    return outp[:, :1]
